# Initial kernel scaffold; baseline (speedup 1.0000x reference)
import jax
import jax.numpy as jnp
from jax.experimental import pallas as pl


def kernel(x, edge_index, batch, edge_weights, W1, b1, W2, b2, W3, b3, W4, b4, W5, b5, W6, b6, W7, b7, W8, b8, Wl, bl):
    raise NotImplementedError("write your pallas kernel here")


# speedup vs baseline: 65.8152x; 65.8152x over previous
import functools

import jax
import jax.numpy as jnp
from jax import lax
from jax.experimental import pallas as pl
from jax.experimental.pallas import tpu as pltpu
from jax.experimental.pallas import tpu_sc as plsc

NC = 2
NS = 16
NW = NC * NS
LANES = 16

N = 100000
NPAD = 100096
E = 3200000
H = 16
G = 64

CHUNK = 128
SUP = 4
SUPE = SUP * CHUNK
NSUPER = E // SUPE
SUP_ITERS = -(-NSUPER // NW)
EROWS = E // CHUNK

ROWS_PER_S = NPAD // NS
ZROWS = 512


def _sc_mesh():
    return plsc.VectorSubcoreMesh(
        core_axis_name="c", subcore_axis_name="s", num_cores=NC, num_subcores=NS
    )


_SC_PARAMS = pltpu.CompilerParams(use_tc_tiling_on_sc=False)


def _sc_spmm(g, srcr, dstr, wr, zeros2):
    @functools.partial(
        pl.kernel,
        out_type=jax.ShapeDtypeStruct((NC * NPAD, H), jnp.float32),
        mesh=_sc_mesh(),
        compiler_params=_SC_PARAMS,
        scratch_types=[
            pltpu.VMEM((SUP, CHUNK), jnp.int32),
            pltpu.VMEM((SUP, CHUNK), jnp.int32),
            pltpu.VMEM((SUP, CHUNK), jnp.int32),
            pltpu.VMEM((SUP, CHUNK), jnp.int32),
            pltpu.VMEM((SUP, CHUNK), jnp.float32),
            pltpu.VMEM((SUP, CHUNK), jnp.float32),
            pltpu.VMEM((SUPE, H), jnp.float32),
            pltpu.VMEM((SUPE, H), jnp.float32),
            pltpu.VMEM_SHARED((NPAD, H), jnp.float32),
            pltpu.SemaphoreType.DMA,
            pltpu.SemaphoreType.DMA,
            pltpu.SemaphoreType.DMA,
            pltpu.SemaphoreType.DMA,
            pltpu.SemaphoreType.DMA,
        ],
    )
    def k(g_hbm, src_hbm, dst_hbm, w_hbm, z_hbm, out_hbm,
          sb0, sb1, db0, db1, wb0, wb1, rb0, rb1, acc,
          semI0, semI1, semG0, semG1, semS):
        c = lax.axis_index("c")
        s = lax.axis_index("s")
        wid = c * NS + s
        sb = (sb0, sb1)
        db = (db0, db1)
        wb = (wb0, wb1)
        rb = (rb0, rb1)
        semI = (semI0, semI1)
        semG = (semG0, semG1)

        pltpu.sync_copy(z_hbm, acc.at[pl.ds(s * ROWS_PER_S, ROWS_PER_S)])
        plsc.subcore_barrier()

        def idx_issue(b, t):
            row0 = (wid + t * NW) * SUP
            pltpu.async_copy(src_hbm.at[pl.ds(row0, SUP)], sb[b], semI[b])
            pltpu.async_copy(dst_hbm.at[pl.ds(row0, SUP)], db[b], semI[b])
            pltpu.async_copy(w_hbm.at[pl.ds(row0, SUP)], wb[b], semI[b])

        def idx_wait(b, t):
            row0 = (wid + t * NW) * SUP
            pltpu.make_async_copy(
                src_hbm.at[pl.ds(row0, SUP)], sb[b], semI[b]).wait()
            pltpu.make_async_copy(
                dst_hbm.at[pl.ds(row0, SUP)], db[b], semI[b]).wait()
            pltpu.make_async_copy(
                w_hbm.at[pl.ds(row0, SUP)], wb[b], semI[b]).wait()

        def gather_issue(b):
            for j in range(SUP):
                pltpu.async_copy(
                    g_hbm.at[sb[b].at[j]],
                    rb[b].at[pl.ds(j * CHUNK, CHUNK)],
                    semG[b],
                )

        def gather_wait(b):
            for j in range(SUP):
                pltpu.make_async_copy(
                    g_hbm.at[sb[b].at[j]],
                    rb[b].at[pl.ds(j * CHUNK, CHUNK)],
                    semG[b],
                ).wait()

        def process(b):
            rows = rb[b]
            for j in range(SUP):
                for tt in range(CHUNK // 16):
                    wg = wb[b][j, pl.ds(tt * 16, 16)]
                    for e in range(16):
                        r = j * CHUNK + tt * 16 + e
                        rows[r, :] = rows[r, :] * jnp.broadcast_to(
                            wg[e], (16,)
                        )
                pltpu.async_copy(
                    rows.at[pl.ds(j * CHUNK, CHUNK)],
                    acc.at[db[b].at[j]],
                    semS,
                    add=True,
                )
            for j in range(SUP):
                pltpu.make_async_copy(
                    rows.at[pl.ds(j * CHUNK, CHUNK)],
                    acc.at[db[b].at[j]],
                    semS,
                ).wait()

        idx_issue(0, 0)
        idx_wait(0, 0)
        gather_issue(0)
        idx_issue(1, 1)

        def body(ip, _):
            for bb in range(2):
                t = ip * 2 + bb
                sup = wid + t * NW

                @pl.when(sup + NW < NSUPER)
                def _():
                    idx_wait(1 - bb, t + 1)
                    gather_issue(1 - bb)

                @pl.when(sup < NSUPER)
                def _():
                    gather_wait(bb)
                    process(bb)

                @pl.when(sup + 2 * NW < NSUPER)
                def _():
                    idx_issue(bb, t + 2)

            return _

        lax.fori_loop(0, SUP_ITERS // 2, body, None)
        plsc.subcore_barrier()
        pltpu.sync_copy(
            acc.at[pl.ds(s * ROWS_PER_S, ROWS_PER_S)],
            out_hbm.at[pl.ds(c * NPAD + s * ROWS_PER_S, ROWS_PER_S)],
        )

    return k(g, srcr, dstr, wr, zeros2)


def _sc_degw(dstr, wr, zeros2):
    @functools.partial(
        pl.kernel,
        out_type=jax.ShapeDtypeStruct((NC * NPAD, H), jnp.float32),
        mesh=_sc_mesh(),
        compiler_params=_SC_PARAMS,
        scratch_types=[
            pltpu.VMEM((SUP, CHUNK), jnp.int32),
            pltpu.VMEM((SUP, CHUNK), jnp.int32),
            pltpu.VMEM((SUP, CHUNK), jnp.float32),
            pltpu.VMEM((SUP, CHUNK), jnp.float32),
            pltpu.VMEM((SUPE, H), jnp.float32),
            pltpu.VMEM_SHARED((NPAD, H), jnp.float32),
            pltpu.SemaphoreType.DMA,
            pltpu.SemaphoreType.DMA,
            pltpu.SemaphoreType.DMA,
        ],
    )
    def k(dst_hbm, w_hbm, z_hbm, out_hbm,
          db0, db1, wb0, wb1, rows, acc, semI0, semI1, semS):
        c = lax.axis_index("c")
        s = lax.axis_index("s")
        wid = c * NS + s
        db = (db0, db1)
        wb = (wb0, wb1)
        semI = (semI0, semI1)

        pltpu.sync_copy(z_hbm, acc.at[pl.ds(s * ROWS_PER_S, ROWS_PER_S)])
        plsc.subcore_barrier()

        def idx_issue(b, t):
            row0 = (wid + t * NW) * SUP
            pltpu.async_copy(dst_hbm.at[pl.ds(row0, SUP)], db[b], semI[b])
            pltpu.async_copy(w_hbm.at[pl.ds(row0, SUP)], wb[b], semI[b])

        def idx_wait(b, t):
            row0 = (wid + t * NW) * SUP
            pltpu.make_async_copy(
                dst_hbm.at[pl.ds(row0, SUP)], db[b], semI[b]).wait()
            pltpu.make_async_copy(
                w_hbm.at[pl.ds(row0, SUP)], wb[b], semI[b]).wait()

        def process(b):
            for j in range(SUP):
                for tt in range(CHUNK // 16):
                    wg = wb[b][j, pl.ds(tt * 16, 16)]
                    for e in range(16):
                        r = j * CHUNK + tt * 16 + e
                        rows[r, :] = jnp.broadcast_to(wg[e], (16,))
                pltpu.async_copy(
                    rows.at[pl.ds(j * CHUNK, CHUNK)],
                    acc.at[db[b].at[j]],
                    semS,
                    add=True,
                )
            for j in range(SUP):
                pltpu.make_async_copy(
                    rows.at[pl.ds(j * CHUNK, CHUNK)],
                    acc.at[db[b].at[j]],
                    semS,
                ).wait()

        idx_issue(0, 0)

        def body(ip, _):
            for bb in range(2):
                t = ip * 2 + bb
                sup = wid + t * NW

                @pl.when(sup + NW < NSUPER)
                def _():
                    idx_issue(1 - bb, t + 1)

                @pl.when(sup < NSUPER)
                def _():
                    idx_wait(bb, t)
                    process(bb)

            return _

        lax.fori_loop(0, SUP_ITERS // 2, body, None)
        plsc.subcore_barrier()
        pltpu.sync_copy(
            acc.at[pl.ds(s * ROWS_PER_S, ROWS_PER_S)],
            out_hbm.at[pl.ds(c * NPAD + s * ROWS_PER_S, ROWS_PER_S)],
        )

    return k(dstr, wr, zeros2)


_POOL_CHUNKS = N // CHUNK
_POOL_TAIL = N - _POOL_CHUNKS * CHUNK
_POOL_ITERS = -(-_POOL_CHUNKS // NW)
TAIL = _POOL_TAIL


def _sc_pool(h, batch):
    @functools.partial(
        pl.kernel,
        out_type=[
            jax.ShapeDtypeStruct((NC * G, H), jnp.float32),
            jax.ShapeDtypeStruct((NC * G,), jnp.float32),
        ],
        mesh=_sc_mesh(),
        compiler_params=_SC_PARAMS,
        scratch_types=[
            pltpu.VMEM((CHUNK,), jnp.int32),
            pltpu.VMEM((CHUNK, H), jnp.float32),
            pltpu.VMEM((CHUNK,), jnp.float32),
            pltpu.VMEM((TAIL,), jnp.int32),
            pltpu.VMEM((TAIL, H), jnp.float32),
            pltpu.VMEM((TAIL,), jnp.float32),
            pltpu.VMEM((G, H), jnp.float32),
            pltpu.VMEM((G,), jnp.float32),
            pltpu.VMEM_SHARED((G, H), jnp.float32),
            pltpu.VMEM_SHARED((G,), jnp.float32),
        ],
    )
    def k(h_hbm, b_hbm, sums_hbm, cnts_hbm,
          b_v, rows_v, ones_v, b_t, rows_t, ones_t, zS, zC, accS, accC):
        c = lax.axis_index("c")
        s = lax.axis_index("s")
        wid = c * NS + s
        one16 = jnp.ones((16,), jnp.float32)
        zero16 = jnp.zeros((16,), jnp.float32)
        for t in range(CHUNK // 16):
            ones_v[pl.ds(t * 16, 16)] = one16
        for t in range(TAIL // 16):
            ones_t[pl.ds(t * 16, 16)] = one16

        @pl.when(s == 0)
        def _():
            for r in range(G):
                zS[r, pl.ds(0, 16)] = zero16
            for t in range(G // 16):
                zC[pl.ds(t * 16, 16)] = zero16
            pltpu.sync_copy(zS, accS)
            pltpu.sync_copy(zC, accC)

        plsc.subcore_barrier()

        def body(i, _):
            chunk = wid + i * NW

            @pl.when(chunk < _POOL_CHUNKS)
            def _():
                base = chunk * CHUNK
                pltpu.sync_copy(h_hbm.at[pl.ds(base, CHUNK)], rows_v)
                pltpu.sync_copy(b_hbm.at[pl.ds(base, CHUNK)], b_v)
                pltpu.sync_copy(rows_v, accS.at[b_v], add=True)
                pltpu.sync_copy(ones_v, accC.at[b_v], add=True)

            return _

        lax.fori_loop(0, _POOL_ITERS, body, None)

        @pl.when(wid == 0)
        def _():
            base = _POOL_CHUNKS * CHUNK
            pltpu.sync_copy(h_hbm.at[pl.ds(base, _POOL_TAIL)], rows_t)
            pltpu.sync_copy(b_hbm.at[pl.ds(base, _POOL_TAIL)], b_t)
            pltpu.sync_copy(rows_t, accS.at[b_t], add=True)
            pltpu.sync_copy(ones_t, accC.at[b_t], add=True)

        plsc.subcore_barrier()

        @pl.when(s == 0)
        def _():
            pltpu.sync_copy(accS, sums_hbm.at[pl.ds(c * G, G)])
            pltpu.sync_copy(accC, cnts_hbm.at[pl.ds(c * G, G)])

    return k(h, batch)


NR = NPAD // 8
_BN = 736
_GRID = NR // _BN

_pk = pl.BlockSpec((_BN, 128), lambda i: (i, 0))
_pk_hi = pl.BlockSpec((_BN, 128), lambda i: (i + _GRID, 0))
_bd = pl.BlockSpec((128, 128), lambda i: (0, 0))
_bb = pl.BlockSpec((8, 128), lambda i: (0, 0))


def _tc_pre(d0d1, xpk, BD1):
    def body(d0_ref, d1_ref, x_ref, w_ref, dinv_ref, g_ref):
        deg = 1.0 + d0_ref[...] + d1_ref[...]
        dinv = lax.rsqrt(deg)
        dinv_ref[...] = dinv
        g_ref[...] = jnp.dot(x_ref[...], w_ref[...],
                             preferred_element_type=jnp.float32) * dinv

    NRX = NR // 4
    bx = _BN // 4
    return pl.pallas_call(
        body,
        grid=(_GRID,),
        in_specs=[
            pl.BlockSpec((bx, 512), lambda i: (i, 0)),
            pl.BlockSpec((bx, 512), lambda i: (i + _GRID, 0)),
            pl.BlockSpec((bx, 128), lambda i: (i, 0)),
            pl.BlockSpec((128, 512), lambda i: (0, 0)),
        ],
        out_specs=(
            pl.BlockSpec((bx, 512), lambda i: (i, 0)),
            pl.BlockSpec((bx, 512), lambda i: (i, 0)),
        ),
        out_shape=(
            jax.ShapeDtypeStruct((NRX, 512), jnp.float32),
            jax.ShapeDtypeStruct((NRX, 512), jnp.float32),
        ),
    )(d0d1, d0d1, xpk, BD1)


def _tc_mid(scat, g, res, dinv, b8, BDn):
    def body(s0_ref, s1_ref, g_ref, res_ref, dinv_ref, b_ref, w_ref,
             res_o_ref, g_o_ref):
        dinv = dinv_ref[...]
        conv = b_ref[0:1, :] + dinv * (s0_ref[...] + s1_ref[...] + g_ref[...])
        res_n = jax.nn.relu(res_ref[...] + conv)
        res_o_ref[...] = res_n
        g_o_ref[...] = jnp.dot(res_n, w_ref[...],
                               preferred_element_type=jnp.float32) * dinv

    return pl.pallas_call(
        body,
        grid=(_GRID,),
        in_specs=[_pk, _pk_hi, _pk, _pk, _pk, _bb, _bd],
        out_specs=(_pk, _pk),
        out_shape=(
            jax.ShapeDtypeStruct((NR, 128), jnp.float32),
            jax.ShapeDtypeStruct((NR, 128), jnp.float32),
        ),
    )(scat, scat, g, res, dinv, b8, BDn)


def _tc_post8(scat, g, res, dinv, b8):
    def body(s0_ref, s1_ref, g_ref, res_ref, dinv_ref, b_ref, h_ref):
        dinv = dinv_ref[...]
        conv = b_ref[0:1, :] + dinv * (s0_ref[...] + s1_ref[...] + g_ref[...])
        h_ref[...] = jax.nn.relu(res_ref[...] + jax.nn.relu(conv))

    return pl.pallas_call(
        body,
        grid=(_GRID,),
        in_specs=[_pk, _pk_hi, _pk, _pk, _pk, _bb],
        out_specs=_pk,
        out_shape=jax.ShapeDtypeStruct((NR, 128), jnp.float32),
    )(scat, scat, g, res, dinv, b8)


def _tc_final(s0, s1, c0, c1, Wlp, blp):
    def body(s0_ref, s1_ref, c0_ref, c1_ref, w_ref, b_ref, o_ref):
        sums = s0_ref[...] + s1_ref[...]
        cnt = jnp.maximum(c0_ref[...] + c1_ref[...], 1.0)
        mean = sums / cnt
        o_ref[...] = jnp.dot(mean, w_ref[...],
                             preferred_element_type=jnp.float32) + b_ref[0:1, :]

    return pl.pallas_call(
        body,
        grid=(1,),
        in_specs=[
            pl.BlockSpec((G, H), lambda i: (0, 0)),
            pl.BlockSpec((G, H), lambda i: (0, 0)),
            pl.BlockSpec((G, 1), lambda i: (0, 0)),
            pl.BlockSpec((G, 1), lambda i: (0, 0)),
            pl.BlockSpec((H, 8), lambda i: (0, 0)),
            pl.BlockSpec((8, 8), lambda i: (0, 0)),
        ],
        out_specs=pl.BlockSpec((G, 8), lambda i: (0, 0)),
        out_shape=jax.ShapeDtypeStruct((G, 8), jnp.float32),
    )(s0, s1, c0, c1, Wlp, blp)


def kernel(x, edge_index, batch, edge_weights, W1, b1, W2, b2, W3, b3, W4, b4,
           W5, b5, W6, b6, W7, b7, W8, b8, Wl, bl):
    src = edge_index[0]
    dst = edge_index[1]
    w = edge_weights

    zeros2 = jnp.zeros((ROWS_PER_S, H), jnp.float32)

    srcr = src.reshape(EROWS, CHUNK)
    dstr = dst.reshape(EROWS, CHUNK)
    wr = w.reshape(EROWS, CHUNK)

    degp = _sc_degw(dstr, wr, zeros2)

    xpk = jnp.pad(x, ((0, NPAD - N), (0, 0))).reshape(NR // 4, 128)
    eye8 = jnp.eye(8, dtype=jnp.float32)
    BD1 = jnp.kron(jnp.eye(32, dtype=jnp.float32), W1)
    dinv_a, g_a = _tc_pre(degp.reshape(2 * (NR // 4), 512), xpk, BD1)
    dinv = dinv_a.reshape(NR, 128)
    g = g_a.reshape(NR, 128)

    res = jnp.zeros((NR, 128), jnp.float32)
    bs = [b1, b2, b3, b4, b5, b6, b7, b8]
    Ws = [W1, W2, W3, W4, W5, W6, W7, W8]
    for l in range(8):
        scat = _sc_spmm(g.reshape(NPAD, H), srcr, dstr, wr, zeros2)
        scat = scat.reshape(2 * NR, 128)
        b128 = jnp.broadcast_to(jnp.tile(bs[l], 8)[None, :], (8, 128))
        if l < 7:
            BDn = jnp.kron(eye8, Ws[l + 1])
            res, g = _tc_mid(scat, g, res, dinv, b128, BDn)
        else:
            h = _tc_post8(scat, g, res, dinv, b128)

    sums, cnts = _sc_pool(h.reshape(NPAD, H), batch)
    Wlp = jnp.pad(Wl, ((0, 0), (0, 8 - Wl.shape[1])))
    blp = jnp.broadcast_to(jnp.pad(bl, (0, 8 - bl.shape[0]))[None, :], (8, 8))
    out = _tc_final(sums[:G], sums[G:], cnts[:G, None], cnts[G:, None],
                    Wlp, blp)
    return out[:, :1]
